# baseline (speedup 1.0000x reference)
import functools

import jax
import jax.numpy as jnp
from jax import lax
from jax.experimental import pallas as pl
from jax.experimental.pallas import tpu as pltpu
from jax.experimental.pallas import tpu_sc as plsc

D = 128
B = 4096
S = 200
TOTAL = B * S
NC = 2
NS = 16
NW = NC * NS
PER_W = TOTAL // NW
C = 128
CH = PER_W // C
NBUF = 5
LEAD = 3


def _build():
  mesh = plsc.VectorSubcoreMesh(core_axis_name="c", subcore_axis_name="s")

  @functools.partial(
      pl.kernel,
      mesh=mesh,
      out_type=[
          jax.ShapeDtypeStruct((TOTAL, D), jnp.float32),
          jax.ShapeDtypeStruct((TOTAL, D), jnp.float32),
      ],
      scratch_types=[
          pltpu.VMEM((CH, C), jnp.int32),
      ] + [pltpu.VMEM((C, D), jnp.float32) for _ in range(NBUF)]
        + [pltpu.SemaphoreType.DMA for _ in range(2 * NBUF)],
  )
  def body(seq_hbm, ts_hbm, tok_hbm, time_hbm, out_tok, out_time,
           idx_v, *bufs_and_sems):
    bufs = bufs_and_sems[:NBUF]
    gsems = bufs_and_sems[NBUF:2 * NBUF]
    wsems = bufs_and_sems[2 * NBUF:]
    wid = lax.axis_index("s") * NC + lax.axis_index("c")
    base = wid * PER_W

    def run(idx_hbm, table, out):
      pltpu.sync_copy(idx_hbm.at[wid], idx_v)

      for b in range(LEAD):
        pltpu.async_copy(table.at[idx_v.at[b]], bufs[b], gsems[b])

      def outer(g, carry):
        for b in range(NBUF):
          j = g * NBUF + b
          f = j + LEAD
          bf = (b + LEAD) % NBUF

          @pl.when((j >= NBUF - LEAD) & (f < CH))
          def _():
            pltpu.make_async_copy(bufs[bf], out.at[pl.ds(0, C)],
                                  wsems[bf]).wait()

          @pl.when(f < CH)
          def _():
            pltpu.async_copy(table.at[idx_v.at[f]], bufs[bf], gsems[bf])

          pltpu.make_async_copy(table.at[idx_v.at[j]], bufs[b],
                                gsems[b]).wait()
          pltpu.async_copy(bufs[b], out.at[pl.ds(base + j * C, C)], wsems[b])
        return carry

      lax.fori_loop(0, CH // NBUF, outer, 0)

      for b in range(NBUF):
        pltpu.make_async_copy(bufs[b], out.at[pl.ds(0, C)], wsems[b]).wait()

    run(seq_hbm, tok_hbm, out_tok)
    run(ts_hbm, time_hbm, out_time)

  return body


_gather = _build()


def kernel(seq, ts, token_table, time_table):
  seq3 = seq.astype(jnp.int32).reshape(NW, CH, C)
  ts3 = ts.astype(jnp.int32).reshape(NW, CH, C)
  out_tok, out_time = _gather(seq3, ts3, token_table, time_table)
  return (out_tok.reshape(B, S, D), out_time.reshape(B, S, D))


# speedup vs baseline: 1.6490x; 1.6490x over previous
import functools

import jax
import jax.numpy as jnp
from jax import lax
from jax.experimental import pallas as pl
from jax.experimental.pallas import tpu as pltpu
from jax.experimental.pallas import tpu_sc as plsc

D = 128
B = 4096
S = 200
TOTAL = B * S
NC = 2
NS = 16
NW = NC * NS
PER_W = TOTAL // NW
C = 128
CH = PER_W // C
NBUF = 5
LEAD = 3


def _build():
  mesh = plsc.VectorSubcoreMesh(core_axis_name="c", subcore_axis_name="s")

  @functools.partial(
      pl.kernel,
      mesh=mesh,
      out_type=[
          jax.ShapeDtypeStruct((TOTAL, D), jnp.float32),
          jax.ShapeDtypeStruct((TOTAL, D), jnp.float32),
      ],
      scratch_types=[
          pltpu.VMEM((CH, C), jnp.int32),
          pltpu.VMEM_SHARED((1440, D), jnp.float32),
      ] + [pltpu.VMEM((C, D), jnp.float32) for _ in range(NBUF)]
        + [pltpu.SemaphoreType.DMA for _ in range(2 * NBUF)],
  )
  def body(seq_hbm, ts_hbm, tok_hbm, time_hbm, out_tok, out_time,
           idx_v, time_sp, *bufs_and_sems):
    bufs = bufs_and_sems[:NBUF]
    gsems = bufs_and_sems[NBUF:2 * NBUF]
    wsems = bufs_and_sems[2 * NBUF:]
    wid = lax.axis_index("s") * NC + lax.axis_index("c")
    base = wid * PER_W

    def run(idx_hbm, table, out):
      pltpu.sync_copy(idx_hbm.at[wid], idx_v)

      for b in range(LEAD):
        pltpu.async_copy(table.at[idx_v.at[b]], bufs[b], gsems[b])

      def outer(g, carry):
        for b in range(NBUF):
          j = g * NBUF + b
          f = j + LEAD
          bf = (b + LEAD) % NBUF

          @pl.when((j >= NBUF - LEAD) & (f < CH))
          def _():
            pltpu.make_async_copy(bufs[bf], out.at[pl.ds(0, C)],
                                  wsems[bf]).wait()

          @pl.when(f < CH)
          def _():
            pltpu.async_copy(table.at[idx_v.at[f]], bufs[bf], gsems[bf])

          pltpu.make_async_copy(table.at[idx_v.at[j]], bufs[b],
                                gsems[b]).wait()
          pltpu.async_copy(bufs[b], out.at[pl.ds(base + j * C, C)], wsems[b])
        return carry

      lax.fori_loop(0, CH // NBUF, outer, 0)

      for b in range(NBUF):
        pltpu.make_async_copy(bufs[b], out.at[pl.ds(0, C)], wsems[b]).wait()

    @pl.when(lax.axis_index("s") == 0)
    def _():
      pltpu.sync_copy(time_hbm, time_sp)

    run(seq_hbm, tok_hbm, out_tok)
    plsc.subcore_barrier()
    run(ts_hbm, time_sp, out_time)

  return body


_gather = _build()


def kernel(seq, ts, token_table, time_table):
  seq3 = seq.astype(jnp.int32).reshape(NW, CH, C)
  ts3 = ts.astype(jnp.int32).reshape(NW, CH, C)
  out_tok, out_time = _gather(seq3, ts3, token_table, time_table)
  return (out_tok.reshape(B, S, D), out_time.reshape(B, S, D))
